# Initial kernel scaffold; baseline (speedup 1.0000x reference)
import functools

import jax
import jax.numpy as jnp
from jax import lax
from jax.experimental import pallas as pl
from jax.experimental.pallas import tpu as pltpu
from jax.experimental.pallas import tpu_sc as plsc

NC = 2
NS = 16
NW = NC * NS

B = 4096 * 200
DIM = 32
CHUNK = 128
B_PER_W = B // NW
N_CHUNKS = B_PER_W // CHUNK


@functools.partial(
    pl.kernel,
    out_type=jax.ShapeDtypeStruct((B, DIM), jnp.float32),
    mesh=plsc.VectorSubcoreMesh(core_axis_name="c", subcore_axis_name="s"),
    scratch_types=[
        pltpu.VMEM((N_CHUNKS, CHUNK), jnp.int32),
        pltpu.VMEM((CHUNK, DIM), jnp.float32),
        pltpu.VMEM((CHUNK, DIM), jnp.float32),
        pltpu.SemaphoreType.DMA,
        pltpu.SemaphoreType.DMA,
    ],
)
def _gather_kernel(table_hbm, xblk_hbm, out_hbm, idx_v, rows_a, rows_b, sem_a, sem_b):
    wid = lax.axis_index("s") * NC + lax.axis_index("c")
    row_base = wid * N_CHUNKS
    out_base = wid * B_PER_W

    pltpu.sync_copy(xblk_hbm.at[pl.ds(row_base, N_CHUNKS)], idx_v)

    bufs = (rows_a, rows_b)
    sems = (sem_a, sem_b)

    pltpu.async_copy(table_hbm.at[idx_v.at[0]], rows_a, sem_a)

    def step(j, carry):
        cur = lax.rem(j, 2)
        for b in range(2):
            @pl.when(jnp.logical_and(cur != b, j + 1 < N_CHUNKS))
            def _():
                pltpu.async_copy(table_hbm.at[idx_v.at[j + 1]], bufs[b], sems[b])
        for b in range(2):
            @pl.when(cur == b)
            def _():
                pltpu.make_async_copy(table_hbm.at[idx_v.at[j]], bufs[b], sems[b]).wait()
                pltpu.sync_copy(bufs[b], out_hbm.at[pl.ds(out_base + j * CHUNK, CHUNK)])
        return carry

    lax.fori_loop(0, N_CHUNKS, step, 0)


def kernel(X, table):
    xblk = X.reshape(B // CHUNK, CHUNK)
    out = _gather_kernel(table, xblk)
    return out.reshape(X.shape[0], X.shape[1], DIM)


# speedup vs baseline: 1.4259x; 1.4259x over previous
import functools

import jax
import jax.numpy as jnp
from jax import lax
from jax.experimental import pallas as pl
from jax.experimental.pallas import tpu as pltpu
from jax.experimental.pallas import tpu_sc as plsc

NC = 2
NS = 16
NW = NC * NS

B = 4096 * 200
DIM = 32
CHUNK = 128
B_PER_W = B // NW
N_CHUNKS = B_PER_W // CHUNK


@functools.partial(
    pl.kernel,
    out_type=jax.ShapeDtypeStruct((B, DIM), jnp.float32),
    mesh=plsc.VectorSubcoreMesh(core_axis_name="c", subcore_axis_name="s"),
    scratch_types=[
        pltpu.VMEM((N_CHUNKS, CHUNK), jnp.int32),
        pltpu.VMEM((CHUNK, DIM), jnp.float32),
        pltpu.VMEM((CHUNK, DIM), jnp.float32),
        pltpu.SemaphoreType.DMA,
        pltpu.SemaphoreType.DMA,
    ],
    compiler_params=pltpu.CompilerParams(use_tc_tiling_on_sc=False),
)
def _gather_kernel(table_hbm, xblk_hbm, out_hbm, idx_v, rows_a, rows_b, sem_a, sem_b):
    wid = lax.axis_index("s") * NC + lax.axis_index("c")
    row_base = wid * N_CHUNKS
    out_base = wid * B_PER_W

    pltpu.sync_copy(xblk_hbm.at[pl.ds(row_base, N_CHUNKS)], idx_v)

    bufs = (rows_a, rows_b)
    sems = (sem_a, sem_b)

    pltpu.async_copy(table_hbm.at[idx_v.at[0]], rows_a, sem_a)

    def step(j, carry):
        cur = lax.rem(j, 2)
        for b in range(2):
            @pl.when(jnp.logical_and(cur != b, j + 1 < N_CHUNKS))
            def _():
                pltpu.async_copy(table_hbm.at[idx_v.at[j + 1]], bufs[b], sems[b])
        for b in range(2):
            @pl.when(cur == b)
            def _():
                pltpu.make_async_copy(table_hbm.at[idx_v.at[j]], bufs[b], sems[b]).wait()
                pltpu.sync_copy(bufs[b], out_hbm.at[pl.ds(out_base + j * CHUNK, CHUNK)])
        return carry

    lax.fori_loop(0, N_CHUNKS, step, 0)


def kernel(X, table):
    xblk = X.reshape(B // CHUNK, CHUNK)
    out = _gather_kernel(table, xblk)
    return out.reshape(X.shape[0], X.shape[1], DIM)


# speedup vs baseline: 1.5011x; 1.0527x over previous
import functools

import jax
import jax.numpy as jnp
from jax import lax
from jax.experimental import pallas as pl
from jax.experimental.pallas import tpu as pltpu
from jax.experimental.pallas import tpu_sc as plsc

NC = 2
NS = 16
NW = NC * NS

B = 4096 * 200
DIM = 32
CHUNK = 1024
B_PER_W = B // NW
N_CHUNKS = B_PER_W // CHUNK


@functools.partial(
    pl.kernel,
    out_type=jax.ShapeDtypeStruct((B, DIM), jnp.float32),
    mesh=plsc.VectorSubcoreMesh(core_axis_name="c", subcore_axis_name="s"),
    scratch_types=[
        pltpu.VMEM((N_CHUNKS, CHUNK), jnp.int32),
        pltpu.VMEM((CHUNK, DIM), jnp.float32),
        pltpu.VMEM((CHUNK, DIM), jnp.float32),
        pltpu.SemaphoreType.DMA,
        pltpu.SemaphoreType.DMA,
    ],
    compiler_params=pltpu.CompilerParams(use_tc_tiling_on_sc=False),
)
def _gather_kernel(table_hbm, xblk_hbm, out_hbm, idx_v, rows_a, rows_b, sem_a, sem_b):
    wid = lax.axis_index("s") * NC + lax.axis_index("c")
    row_base = wid * N_CHUNKS
    out_base = wid * B_PER_W

    pltpu.sync_copy(xblk_hbm.at[pl.ds(row_base, N_CHUNKS)], idx_v)

    bufs = (rows_a, rows_b)
    sems = (sem_a, sem_b)

    pltpu.async_copy(table_hbm.at[idx_v.at[0]], rows_a, sem_a)

    def step(j, carry):
        cur = lax.rem(j, 2)
        for b in range(2):
            @pl.when(jnp.logical_and(cur != b, j + 1 < N_CHUNKS))
            def _():
                pltpu.async_copy(table_hbm.at[idx_v.at[j + 1]], bufs[b], sems[b])
        for b in range(2):
            @pl.when(cur == b)
            def _():
                pltpu.make_async_copy(table_hbm.at[idx_v.at[j]], bufs[b], sems[b]).wait()
                pltpu.sync_copy(bufs[b], out_hbm.at[pl.ds(out_base + j * CHUNK, CHUNK)])
        return carry

    lax.fori_loop(0, N_CHUNKS, step, 0)


def kernel(X, table):
    xblk = X.reshape(B // CHUNK, CHUNK)
    out = _gather_kernel(table, xblk)
    return out.reshape(X.shape[0], X.shape[1], DIM)
